# baseline (speedup 1.0000x reference)
import jax
import jax.numpy as jnp
from jax.experimental import pallas as pl
from jax.experimental.pallas import tpu as pltpu

_BLK = 64
_K = 3


def _make(batch, dim):
    nstep = batch // _BLK

    def body(e_ref, out_ref, ebuf, bufs, sems):
        cp = pltpu.make_async_copy(e_ref, ebuf, sems.at[_K])
        cp.start()
        cp.wait()
        i = jax.lax.broadcasted_iota(jnp.int32, (dim, dim), 0)
        j = jax.lax.broadcasted_iota(jnp.int32, (dim, dim), 1)
        eye = jnp.where(i == j, jnp.float32(1), jnp.float32(0))
        handles = [None] * _K
        for s in range(nstep):
            k = s % _K
            if handles[k] is not None:
                handles[k].wait()
            vals = jnp.exp(ebuf[pl.ds(s * _BLK, _BLK), :])
            bufs[k] = vals[:, None, :] * eye[None, :, :]
            h = pltpu.make_async_copy(
                bufs.at[k], out_ref.at[pl.ds(s * _BLK, _BLK)], sems.at[k])
            h.start()
            handles[k] = h
        for h in handles:
            if h is not None:
                h.wait()

    return pl.pallas_call(
        body,
        in_specs=[pl.BlockSpec(memory_space=pl.ANY)],
        out_specs=pl.BlockSpec(memory_space=pl.ANY),
        out_shape=jax.ShapeDtypeStruct((batch, dim, dim), jnp.float32),
        scratch_shapes=[
            pltpu.VMEM((batch, dim), jnp.float32),
            pltpu.VMEM((_K, _BLK, dim, dim), jnp.float32),
            pltpu.SemaphoreType.DMA((_K + 1,)),
        ],
    )


def kernel(embeddings):
    batch, dim, _ = embeddings.shape
    e2 = embeddings[:, :, 0]
    return _make(batch, dim)(e2)
